# Initial kernel scaffold; baseline (speedup 1.0000x reference)
import jax
import jax.numpy as jnp
from jax.experimental import pallas as pl


def kernel(x, batch, params):
    raise NotImplementedError("write your pallas kernel here")


# speedup vs baseline: 6.1095x; 6.1095x over previous
import functools

import jax
import jax.numpy as jnp
from jax import lax
from jax.experimental import pallas as pl
from jax.experimental.pallas import tpu as pltpu
from jax.experimental.pallas import tpu_sc as plsc

N = 16384
K = 16
EPS = 1e-5
GP = 128


def _bf(v):
    return v.astype(jnp.bfloat16)


BQ = 256
BK = 512


def _knn_body(xq_ref, xt_ref, q2_ref, k2_ref, bcol_ref, brow_ref, idx_ref):
    Xq = _bf(xq_ref[...])
    q2 = q2_ref[...]
    bq = bcol_ref[...]
    bfirst = bcol_ref[0, 0]
    blast = bcol_ref[BQ - 1, 0]
    brow = brow_ref[...]
    lo = jnp.sum((brow < bfirst).astype(jnp.int32))
    hi = jnp.sum((brow <= blast).astype(jnp.int32))
    clo = lo // BK
    chi = (hi + BK - 1) // BK

    def chunk_body(c, carry):
        cand_d, cand_i = carry
        koff = pl.multiple_of(c * BK, BK)
        XcT = xt_ref[:, pl.ds(koff, BK)]
        s = jnp.dot(Xq, _bf(XcT), preferred_element_type=jnp.float32)
        d2 = (q2 - 2.0 * s) + k2_ref[:, pl.ds(koff, BK)]
        bk = brow_ref[:, pl.ds(koff, BK)]
        d2 = jnp.where(bq != bk, jnp.inf, d2)
        colid = koff + lax.broadcasted_iota(jnp.int32, (BQ, BK), 1)
        lane = lax.broadcasted_iota(jnp.int32, (BQ, K), 1)

        def ext(_, c2):
            d2c, cd, ci = c2
            m = jnp.min(d2c, axis=1, keepdims=True)
            am = jnp.min(jnp.where(d2c <= m, colid, N),
                         axis=1, keepdims=True)
            T = jnp.max(cd, axis=1, keepdims=True)
            ws = jnp.min(jnp.where(cd >= T, lane, K),
                         axis=1, keepdims=True)
            upd = (lane == ws) & (m < T)
            cd = jnp.where(upd, m, cd)
            ci = jnp.where(upd, am, ci)
            d2c = jnp.where(colid == am, jnp.inf, d2c)
            return d2c, cd, ci

        _, cand_d, cand_i = lax.fori_loop(0, K, ext, (d2, cand_d, cand_i))
        return cand_d, cand_i

    cand_d = jnp.full((BQ, K), jnp.inf, jnp.float32)
    cand_i = jnp.zeros((BQ, K), jnp.int32)
    _, cand_i = lax.fori_loop(clo, chi, chunk_body, (cand_d, cand_i))
    idx_ref[...] = cand_i


def _knn(x, x2, bcol, brow):
    n, d = x.shape
    return pl.pallas_call(
        _knn_body,
        grid=(n // BQ,),
        in_specs=[
            pl.BlockSpec((BQ, d), lambda i: (i, 0)),
            pl.BlockSpec((d, n), lambda i: (0, 0)),
            pl.BlockSpec((BQ, 1), lambda i: (i, 0)),
            pl.BlockSpec((1, n), lambda i: (0, 0)),
            pl.BlockSpec((BQ, 1), lambda i: (i, 0)),
            pl.BlockSpec((1, n), lambda i: (0, 0)),
        ],
        out_specs=pl.BlockSpec((BQ, K), lambda i: (i, 0)),
        out_shape=jax.ShapeDtypeStruct((n, K), jnp.int32),
    )(x, x.T, x2.reshape(n, 1), x2.reshape(1, n), bcol, brow)


_NC, _NS = 2, 16
_NW = _NC * _NS
_GCH = 128


@jax.jit
def _sc_gather(table, idx):
    perw = (N * K) // _NW
    mesh = plsc.VectorSubcoreMesh(core_axis_name="c", subcore_axis_name="s")

    @functools.partial(
        pl.kernel,
        mesh=mesh,
        out_type=jax.ShapeDtypeStruct((N * K, GP), jnp.float32),
        scratch_types=[
            pltpu.VMEM((_GCH,), jnp.int32),
            pltpu.VMEM((_GCH, GP), jnp.float32),
            pltpu.SemaphoreType.DMA,
        ],
    )
    def gk(table_hbm, idx_hbm, out_hbm, idx_v, rows_v, sem):
        wid = lax.axis_index("s") * _NC + lax.axis_index("c")
        base = wid * perw

        def step(j, carry):
            off = base + j * _GCH
            pltpu.sync_copy(idx_hbm.at[pl.ds(off, _GCH)], idx_v)
            pltpu.async_copy(table_hbm.at[idx_v], rows_v, sem).wait()
            pltpu.sync_copy(rows_v, out_hbm.at[pl.ds(off, _GCH)])
            return carry

        lax.fori_loop(0, perw // _GCH, step, 0)

    return gk(table, idx)


BD = 128


def _h1s_body(g_ref, x_ref, w_ref, b_ref, h1_ref):
    d = x_ref.shape[1]
    xi = x_ref[...]
    xj = g_ref[...][:, :, :d]
    xi3 = jnp.broadcast_to(xi[:, None, :], (BD, K, d))
    msg = jnp.concatenate([xi3.reshape(BD * K, d),
                           (xj - xi[:, None, :]).reshape(BD * K, d)], axis=1)
    mm = jnp.dot(_bf(msg), _bf(w_ref[...]),
                 preferred_element_type=jnp.float32) + b_ref[...]
    h1_ref[...] = mm.reshape(BD, K, mm.shape[-1])


def _h1s(g3, x, W1, b1):
    n, d = x.shape
    c1 = W1.shape[1]
    return pl.pallas_call(
        _h1s_body,
        grid=(n // BD,),
        in_specs=[
            pl.BlockSpec((BD, K, GP), lambda i: (i, 0, 0)),
            pl.BlockSpec((BD, d), lambda i: (i, 0)),
            pl.BlockSpec((2 * d, c1), lambda i: (0, 0)),
            pl.BlockSpec((1, c1), lambda i: (0, 0)),
        ],
        out_specs=pl.BlockSpec((BD, K, c1), lambda i: (i, 0, 0)),
        out_shape=jax.ShapeDtypeStruct((n, K, c1), jnp.float32),
    )(g3, x, W1, b1.reshape(1, c1))


def _bn_apply(h, mu_ref, var_ref, g_ref, be_ref):
    sh = h.shape[-1]
    return (g_ref[...].reshape(1, 1, sh) * (h - mu_ref[...].reshape(1, 1, sh))
            * lax.rsqrt(var_ref[...].reshape(1, 1, sh) + EPS)
            + be_ref[...].reshape(1, 1, sh))


def _midh_body(h_ref, mu_ref, var_ref, bg_ref, bb_ref, w_ref, b_ref, h2_ref):
    y = jnp.maximum(_bn_apply(h_ref[...], mu_ref, var_ref, bg_ref, bb_ref),
                    0.0)
    y2 = y.reshape(BD * K, y.shape[-1])
    mm = jnp.dot(_bf(y2), _bf(w_ref[...]),
                 preferred_element_type=jnp.float32) + b_ref[...]
    h2_ref[...] = mm.reshape(BD, K, mm.shape[-1])


def _midh(h3, mu, var, bn_g, bn_be, W, b):
    n, _, c1 = h3.shape
    c2 = W.shape[1]
    nb = n // BD
    return pl.pallas_call(
        _midh_body,
        grid=(nb,),
        in_specs=[
            pl.BlockSpec((BD, K, c1), lambda i: (i, 0, 0)),
            pl.BlockSpec((1, c1), lambda i: (0, 0)),
            pl.BlockSpec((1, c1), lambda i: (0, 0)),
            pl.BlockSpec((1, c1), lambda i: (0, 0)),
            pl.BlockSpec((1, c1), lambda i: (0, 0)),
            pl.BlockSpec((c1, c2), lambda i: (0, 0)),
            pl.BlockSpec((1, c2), lambda i: (0, 0)),
        ],
        out_specs=pl.BlockSpec((BD, K, c2), lambda i: (i, 0, 0)),
        out_shape=jax.ShapeDtypeStruct((n, K, c2), jnp.float32),
    )(h3, mu.reshape(1, c1), var.reshape(1, c1), bn_g.reshape(1, c1),
      bn_be.reshape(1, c1), W, b.reshape(1, c2))


def _fin_body(h_ref, mu_ref, var_ref, bg_ref, bb_ref, w_ref, b_ref, o_ref):
    y = jnp.maximum(_bn_apply(h_ref[...], mu_ref, var_ref, bg_ref, bb_ref),
                    0.0)
    y2 = y.reshape(BD * K, y.shape[-1])
    mm = jnp.dot(_bf(y2), _bf(w_ref[...]), preferred_element_type=jnp.float32)
    mm = mm.reshape(BD, K, mm.shape[-1])
    o_ref[...] = jnp.max(mm, axis=1) + b_ref[...]


def _fin(h3, mu, var, bn_g, bn_be, W, b):
    n, _, c1 = h3.shape
    c2 = W.shape[1]
    return pl.pallas_call(
        _fin_body,
        grid=(n // BD,),
        in_specs=[
            pl.BlockSpec((BD, K, c1), lambda i: (i, 0, 0)),
            pl.BlockSpec((1, c1), lambda i: (0, 0)),
            pl.BlockSpec((1, c1), lambda i: (0, 0)),
            pl.BlockSpec((1, c1), lambda i: (0, 0)),
            pl.BlockSpec((1, c1), lambda i: (0, 0)),
            pl.BlockSpec((c1, c2), lambda i: (0, 0)),
            pl.BlockSpec((1, c2), lambda i: (0, 0)),
        ],
        out_specs=pl.BlockSpec((BD, c2), lambda i: (i, 0)),
        out_shape=jax.ShapeDtypeStruct((n, c2), jnp.float32),
    )(h3, mu.reshape(1, c1), var.reshape(1, c1), bn_g.reshape(1, c1),
      bn_be.reshape(1, c1), W, b.reshape(1, c2))


BH = 512


def _head_body(f1_ref, f2_ref, f3_ref, w1_ref, b1_ref, w2_ref, b2_ref,
               w3_ref, b3_ref, o_ref):
    h = jnp.concatenate([f1_ref[...], f2_ref[...], f3_ref[...]], axis=1)
    h = jnp.maximum(
        jnp.dot(_bf(h), _bf(w1_ref[...]), preferred_element_type=jnp.float32)
        + b1_ref[...], 0.0)
    h = jnp.maximum(
        jnp.dot(_bf(h), _bf(w2_ref[...]), preferred_element_type=jnp.float32)
        + b2_ref[...], 0.0)
    o_ref[...] = jnp.dot(_bf(h), _bf(w3_ref[...]),
                         preferred_element_type=jnp.float32) + b3_ref[...]


def _head(f1, f2, f3, hp):
    n = f1.shape[0]
    ncls = hp['W3'].shape[1]
    return pl.pallas_call(
        _head_body,
        grid=(n // BH,),
        in_specs=[
            pl.BlockSpec((BH, f1.shape[1]), lambda i: (i, 0)),
            pl.BlockSpec((BH, f2.shape[1]), lambda i: (i, 0)),
            pl.BlockSpec((BH, f3.shape[1]), lambda i: (i, 0)),
            pl.BlockSpec(hp['W1'].shape, lambda i: (0, 0)),
            pl.BlockSpec((1, hp['W1'].shape[1]), lambda i: (0, 0)),
            pl.BlockSpec(hp['W2'].shape, lambda i: (0, 0)),
            pl.BlockSpec((1, hp['W2'].shape[1]), lambda i: (0, 0)),
            pl.BlockSpec(hp['W3'].shape, lambda i: (0, 0)),
            pl.BlockSpec((1, ncls), lambda i: (0, 0)),
        ],
        out_specs=pl.BlockSpec((BH, ncls), lambda i: (i, 0)),
        out_shape=jax.ShapeDtypeStruct((n, ncls), jnp.float32),
    )(f1, f2, f3,
      hp['W1'], hp['b1'].reshape(1, -1),
      hp['W2'], hp['b2'].reshape(1, -1),
      hp['W3'], hp['b3'].reshape(1, -1))


def _gathered(x, bcol, brow):
    n, d = x.shape
    x2 = jnp.sum(x * x, axis=1)
    idx = _knn(x, x2, bcol, brow)
    xpad = x if d == GP else jnp.pad(x, ((0, 0), (0, GP - d)))
    g = _sc_gather(xpad, idx.reshape(N * K))
    return g.reshape(N, K, GP)


def _stats(h3):
    h2d = h3.reshape(N * K, h3.shape[2])
    return jnp.mean(h2d, axis=0), jnp.var(h2d, axis=0)


def _edge_conv2(x, bcol, brow, p):
    g3 = _gathered(x, bcol, brow)
    h1 = _h1s(g3, x, p['W1'], p['b1'])
    mu, var = _stats(h1)
    return _fin(h1, mu, var, p['g1'], p['be1'], p['W2'], p['b2'])


def _edge_conv3(x, bcol, brow, p):
    g3 = _gathered(x, bcol, brow)
    h1 = _h1s(g3, x, p['W1'], p['b1'])
    mu, var = _stats(h1)
    h2 = _midh(h1, mu, var, p['g1'], p['be1'], p['W2'], p['b2'])
    mu2, var2 = _stats(h2)
    return _fin(h2, mu2, var2, p['g2'], p['be2'], p['W3'], p['b3'])


def kernel(x, batch, params):
    bcol = batch.reshape(N, 1)
    brow = batch.reshape(1, N)
    f1 = _edge_conv3(x, bcol, brow, params['ec1'])
    f2 = _edge_conv2(f1, bcol, brow, params['ec2'])
    f3 = _edge_conv2(f2, bcol, brow, params['ec3'])
    return _head(f1, f2, f3, params['head'])
